# Initial kernel scaffold; baseline (speedup 1.0000x reference)
import jax
import jax.numpy as jnp
from jax.experimental import pallas as pl


def kernel(x, edge_index, edge_weight, W0_0, W1_0, b_0, W0_1, W1_1, b_1, W0_2, W1_2, b_2, W0_3, W1_3, b_3):
    raise NotImplementedError("write your pallas kernel here")


# speedup vs baseline: 11.1144x; 11.1144x over previous
import functools
import math

import jax
import jax.numpy as jnp
from jax import lax
from jax.experimental import pallas as pl
from jax.experimental.pallas import tpu as pltpu
from jax.experimental.pallas import tpu_sc as plsc

_NC = 2
_NS = 16
_NW = _NC * _NS
_C = 320
_CD = 640


def _sc_mesh():
    return plsc.VectorSubcoreMesh(core_axis_name="c", subcore_axis_name="s")


def _pad_edges(row, col, w):
    e = row.shape[0]
    per_w = math.ceil(e / (_NW * _CD)) * _CD
    ch = per_w // _C
    chd = per_w // _CD
    pad = per_w * _NW - e
    row = jnp.concatenate([row, jnp.zeros((pad,), jnp.int32)])
    col = jnp.concatenate([col, jnp.zeros((pad,), jnp.int32)])
    w = jnp.concatenate([w, jnp.zeros((pad,), jnp.float32)])
    pk = jnp.stack([row.reshape(_NW, ch, _C),
                    col.reshape(_NW, ch, _C),
                    lax.bitcast_convert_type(w, jnp.int32).reshape(_NW, ch, _C)],
                   axis=2)
    return (row.reshape(_NW, chd, _CD), w.reshape(_NW, chd, _CD), chd,
            pk, ch)


def _deg_kernel(row3, w3, n_pad, chd):
    spr = n_pad // _NS

    @functools.partial(
        pl.kernel,
        out_type=jax.ShapeDtypeStruct((_NC, n_pad), jnp.float32),
        mesh=_sc_mesh(),
        scratch_types=[
            pltpu.VMEM((_CD,), jnp.int32),
            pltpu.VMEM((_CD,), jnp.float32),
            pltpu.VMEM((spr,), jnp.float32),
            pltpu.VMEM_SHARED((n_pad,), jnp.float32),
        ],
        compiler_params=pltpu.CompilerParams(use_tc_tiling_on_sc=False),
    )
    def deg(row_h, w_h, out_h, rowv, wv, zv, acc):
        cid = lax.axis_index("c")
        sid = lax.axis_index("s")
        wid = cid * _NS + sid

        def zero(i, carry):
            zv[pl.ds(i * 16, 16)] = jnp.zeros((16,), jnp.float32)
            return carry

        lax.fori_loop(0, spr // 16, zero, 0)
        pltpu.sync_copy(zv, acc.at[pl.ds(sid * spr, spr)])
        plsc.subcore_barrier()

        def chunk(c, carry):
            pltpu.sync_copy(row_h.at[wid, c], rowv)
            pltpu.sync_copy(w_h.at[wid, c], wv)
            pltpu.sync_copy(wv, acc.at[rowv], add=True)
            return carry

        lax.fori_loop(0, chd, chunk, 0)
        plsc.subcore_barrier()
        pltpu.sync_copy(acc.at[pl.ds(sid * spr, spr)],
                        out_h.at[cid, pl.ds(sid * spr, spr)])

    return deg(row3, w3)


def _scatter_kernel(tab, pk, n_pad, ch):
    dp = tab.shape[1]
    d = dp * 2
    spr = n_pad // _NS

    @functools.partial(
        pl.kernel,
        out_type=jax.ShapeDtypeStruct((_NC, n_pad, d), jnp.float32),
        mesh=_sc_mesh(),
        scratch_types=[
            pltpu.VMEM((2, 3, _C), jnp.int32),
            pltpu.VMEM((2, _C, dp), jnp.int32),
            pltpu.VMEM((2, _C, d), jnp.float32),
            pltpu.VMEM_SHARED((n_pad, d), jnp.float32),
            pltpu.SemaphoreType.DMA,
            pltpu.SemaphoreType.DMA,
        ],
        compiler_params=pltpu.CompilerParams(use_tc_tiling_on_sc=False,
                                             needs_layout_passes=False),
    )
    def scat(tab_h, pk_h, out_h, ibuf, rbuf, rows, acc, sem_g, sem_s):
        cid = lax.axis_index("c")
        sid = lax.axis_index("s")
        wid = cid * _NS + sid

        def zero(i, carry):
            for k in range(d // 16):
                rows[0, i, pl.ds(k * 16, 16)] = jnp.zeros((16,), jnp.float32)
            return carry

        lax.fori_loop(0, _C, zero, 0)

        def zcopy(i, carry):
            pltpu.sync_copy(rows.at[0],
                            acc.at[pl.ds(sid * spr + i * _C, _C)])
            return carry

        lax.fori_loop(0, spr // _C, zcopy, 0)
        plsc.subcore_barrier()

        mask_hi = jnp.full((16,), -65536, jnp.int32)

        def scale(b):
            def body(g, carry):
                w_vec = plsc.bitcast(ibuf[b, 2, pl.ds(g * 16, 16)], jnp.float32)
                for e in range(16):
                    r = g * 16 + e
                    s = w_vec[e]
                    for k in range(dp // 16):
                        x = rbuf[b, r, pl.ds(k * 16, 16)]
                        lo = plsc.bitcast(lax.shift_left(x, 16), jnp.float32)
                        hi = plsc.bitcast(lax.bitwise_and(x, mask_hi),
                                          jnp.float32)
                        rows[b, r, pl.ds(k * 16, 16)] = lo * s
                        rows[b, r, pl.ds(dp + k * 16, 16)] = hi * s
                return carry

            lax.fori_loop(0, _C // 16, body, 0)

        descs_s = [None] * ch
        pltpu.sync_copy(pk_h.at[wid, 0], ibuf.at[0])
        desc_g = pltpu.async_copy(tab_h.at[ibuf.at[0, 0]], rbuf.at[0], sem_g)
        for c in range(ch):
            b = c % 2
            nb = 1 - b
            next_g = None
            if c + 1 < ch:
                if c >= 1:
                    descs_s[c - 1].wait()
                pltpu.sync_copy(pk_h.at[wid, c + 1], ibuf.at[nb])
                next_g = pltpu.async_copy(tab_h.at[ibuf.at[nb, 0]],
                                          rbuf.at[nb], sem_g)
            desc_g.wait()
            scale(b)
            descs_s[c] = pltpu.async_copy(rows.at[b], acc.at[ibuf.at[b, 1]],
                                          sem_s, add=True)
            desc_g = next_g
        if ch >= 2:
            descs_s[ch - 2].wait()
        descs_s[ch - 1].wait()
        plsc.subcore_barrier()
        pltpu.sync_copy(acc.at[pl.ds(sid * spr, spr)],
                        out_h.at[cid, pl.ds(sid * spr, spr)])

    return scat(tab, pk)


_B = 2000


def _tc_pre(degt, x, w1):
    n, din = x.shape
    d = w1.shape[1]

    def body(deg_ref, x_ref, w1_ref, dis_ref, ps_ref):
        deg = deg_ref[:, 0] + deg_ref[:, 1]
        dis = jnp.where(deg > 0, lax.rsqrt(deg), 0.0)[:, None]
        dis_ref[:] = dis
        ps_ref[:] = dis * jnp.dot(x_ref[:], w1_ref[:],
                                  preferred_element_type=jnp.float32)

    return pl.pallas_call(
        body,
        grid=(n // _B,),
        in_specs=[
            pl.BlockSpec((_B, 2), lambda i: (i, 0)),
            pl.BlockSpec((_B, din), lambda i: (i, 0)),
            pl.BlockSpec((din, d), lambda i: (0, 0)),
        ],
        out_specs=[
            pl.BlockSpec((_B, 1), lambda i: (i, 0)),
            pl.BlockSpec((_B, d), lambda i: (i, 0)),
        ],
        out_shape=[
            jax.ShapeDtypeStruct((n, 1), jnp.float32),
            jax.ShapeDtypeStruct((n, d), jnp.float32),
        ],
    )(degt, x, w1)


def _tc_layer(h, a2, dis, w0, b, w1n):
    n, din = h.shape
    dout = w0.shape[1]
    ds = a2.shape[2]

    def body(h_ref, a_ref, dis_ref, w0_ref, b_ref, *rest):
        if w1n is None:
            (h1_ref, ps_ref) = rest
        else:
            (w1n_ref, h1_ref, ps_ref) = rest
        dis_b = dis_ref[:]
        m = -(dis_b * (a_ref[0, :, :] + a_ref[1, :, :]))
        z = jnp.dot(h_ref[:], w0_ref[:], preferred_element_type=jnp.float32)
        h1 = 1.0 / (1.0 + jnp.exp(-(z + m + b_ref[:])))
        h1_ref[:] = h1
        if w1n is None:
            ps_ref[:] = dis_b * h1
        else:
            ps_ref[:] = dis_b * jnp.dot(h1, w1n_ref[:],
                                        preferred_element_type=jnp.float32)

    in_specs = [
        pl.BlockSpec((_B, din), lambda i: (i, 0)),
        pl.BlockSpec((2, _B, ds), lambda i: (0, i, 0)),
        pl.BlockSpec((_B, 1), lambda i: (i, 0)),
        pl.BlockSpec((din, dout), lambda i: (0, 0)),
        pl.BlockSpec((1, dout), lambda i: (0, 0)),
    ]
    args = [h, a2, dis, w0, b.reshape(1, -1)]
    if w1n is not None:
        in_specs.append(pl.BlockSpec(w1n.shape, lambda i: (0, 0)))
        args.append(w1n)
    psd = ds if w1n is None else w1n.shape[1]
    return pl.pallas_call(
        body,
        grid=(n // _B,),
        in_specs=in_specs,
        out_specs=[
            pl.BlockSpec((_B, dout), lambda i: (i, 0)),
            pl.BlockSpec((_B, psd), lambda i: (i, 0)),
        ],
        out_shape=[
            jax.ShapeDtypeStruct((n, dout), jnp.float32),
            jax.ShapeDtypeStruct((n, psd), jnp.float32),
        ],
    )(*args)


def _tc_final(h, s2, dis, w0, b, w1):
    n, din = h.shape
    dout = w0.shape[1]
    ds = s2.shape[2]

    def body(h_ref, s_ref, dis_ref, w0_ref, b_ref, w1_ref, out_ref):
        t = -(dis_ref[:] * (s_ref[0, :, :] + s_ref[1, :, :]))
        z = (jnp.dot(h_ref[:], w0_ref[:], preferred_element_type=jnp.float32)
             + jnp.dot(t, w1_ref[:], preferred_element_type=jnp.float32)
             + b_ref[:])
        out_ref[:] = 1.0 / (1.0 + jnp.exp(-z))

    return pl.pallas_call(
        body,
        grid=(n // _B,),
        in_specs=[
            pl.BlockSpec((_B, din), lambda i: (i, 0)),
            pl.BlockSpec((2, _B, ds), lambda i: (0, i, 0)),
            pl.BlockSpec((_B, 1), lambda i: (i, 0)),
            pl.BlockSpec((din, dout), lambda i: (0, 0)),
            pl.BlockSpec((1, dout), lambda i: (0, 0)),
            pl.BlockSpec((ds, dout), lambda i: (0, 0)),
        ],
        out_specs=pl.BlockSpec((_B, dout), lambda i: (i, 0)),
        out_shape=jax.ShapeDtypeStruct((n, dout), jnp.float32),
    )(h, s2, dis, w0, b.reshape(1, -1), w1)


def kernel(x, edge_index, edge_weight,
           W0_0, W1_0, b_0,
           W0_1, W1_1, b_1,
           W0_2, W1_2, b_2,
           W0_3, W1_3, b_3):
    n = x.shape[0]
    row = edge_index[0]
    col = edge_index[1]
    row3, w3, chd, pk, ch = _pad_edges(row, col, edge_weight)
    spr = math.ceil(n / (_NS * _C)) * _C
    n_pad = spr * _NS

    def padn(t):
        bf = t.astype(jnp.bfloat16)
        d = t.shape[1]
        packed = lax.bitcast_convert_type(
            jnp.stack([bf[:, :d // 2], bf[:, d // 2:]], axis=-1), jnp.int32)
        return jnp.pad(packed, ((0, n_pad - n), (0, 0)))

    degt = _deg_kernel(row3, w3, n_pad, chd)[:, :n].T
    dis, ps = _tc_pre(degt, x, W1_0)

    a2 = _scatter_kernel(padn(ps), pk, n_pad, ch)[:, :n, :]
    h, ps = _tc_layer(x, a2, dis, W0_0, b_0, W1_1)

    a2 = _scatter_kernel(padn(ps), pk, n_pad, ch)[:, :n, :]
    h, ps = _tc_layer(h, a2, dis, W0_1, b_1, W1_2)

    a2 = _scatter_kernel(padn(ps), pk, n_pad, ch)[:, :n, :]
    h, ps = _tc_layer(h, a2, dis, W0_2, b_2, None)

    s2 = _scatter_kernel(padn(ps), pk, n_pad, ch)[:, :n, :]
    return _tc_final(h, s2, dis, W0_3, b_3, W1_3)


# speedup vs baseline: 11.9801x; 1.0779x over previous
import functools
import math

import jax
import jax.numpy as jnp
from jax import lax
from jax.experimental import pallas as pl
from jax.experimental.pallas import tpu as pltpu
from jax.experimental.pallas import tpu_sc as plsc

_NC = 2
_NS = 16
_NW = _NC * _NS
_C = 320
_CD = 640


def _sc_mesh():
    return plsc.VectorSubcoreMesh(core_axis_name="c", subcore_axis_name="s")


def _pad_edges(row, col, w):
    e = row.shape[0]
    per_w = math.ceil(e / (_NW * _CD)) * _CD
    ch = per_w // _C
    chd = per_w // _CD
    pad = per_w * _NW - e
    row = jnp.concatenate([row, jnp.zeros((pad,), jnp.int32)])
    col = jnp.concatenate([col, jnp.zeros((pad,), jnp.int32)])
    w = jnp.concatenate([w, jnp.zeros((pad,), jnp.float32)])
    pk = jnp.stack([row.reshape(_NW, ch, _C),
                    col.reshape(_NW, ch, _C),
                    lax.bitcast_convert_type(w, jnp.int32).reshape(_NW, ch, _C)],
                   axis=2)
    return (row.reshape(_NW, chd, _CD), w.reshape(_NW, chd, _CD), chd,
            pk, ch)


def _deg_kernel(row3, w3, n_pad, chd):
    spr = n_pad // _NS

    @functools.partial(
        pl.kernel,
        out_type=jax.ShapeDtypeStruct((_NC, n_pad), jnp.float32),
        mesh=_sc_mesh(),
        scratch_types=[
            pltpu.VMEM((_CD,), jnp.int32),
            pltpu.VMEM((_CD,), jnp.float32),
            pltpu.VMEM((spr,), jnp.float32),
            pltpu.VMEM_SHARED((n_pad,), jnp.float32),
        ],
        compiler_params=pltpu.CompilerParams(use_tc_tiling_on_sc=False),
    )
    def deg(row_h, w_h, out_h, rowv, wv, zv, acc):
        cid = lax.axis_index("c")
        sid = lax.axis_index("s")
        wid = cid * _NS + sid

        def zero(i, carry):
            zv[pl.ds(i * 16, 16)] = jnp.zeros((16,), jnp.float32)
            return carry

        lax.fori_loop(0, spr // 16, zero, 0)
        pltpu.sync_copy(zv, acc.at[pl.ds(sid * spr, spr)])
        plsc.subcore_barrier()

        def chunk(c, carry):
            pltpu.sync_copy(row_h.at[wid, c], rowv)
            pltpu.sync_copy(w_h.at[wid, c], wv)
            pltpu.sync_copy(wv, acc.at[rowv], add=True)
            return carry

        lax.fori_loop(0, chd, chunk, 0)
        plsc.subcore_barrier()
        pltpu.sync_copy(acc.at[pl.ds(sid * spr, spr)],
                        out_h.at[cid, pl.ds(sid * spr, spr)])

    return deg(row3, w3)


def _scatter_kernel(tab, pk, n_pad, ch):
    dp = tab.shape[1]
    d = dp * 2
    spr = n_pad // _NS

    @functools.partial(
        pl.kernel,
        out_type=jax.ShapeDtypeStruct((_NC, n_pad, d), jnp.float32),
        mesh=_sc_mesh(),
        scratch_types=[
            pltpu.VMEM((3, 3, _C), jnp.int32),
            pltpu.VMEM((2, _C, dp), jnp.int32),
            pltpu.VMEM((3, _C, d), jnp.float32),
            pltpu.VMEM_SHARED((n_pad, d), jnp.float32),
            pltpu.SemaphoreType.DMA,
            pltpu.SemaphoreType.DMA,
        ],
        compiler_params=pltpu.CompilerParams(use_tc_tiling_on_sc=False,
                                             needs_layout_passes=False),
    )
    def scat(tab_h, pk_h, out_h, ibuf, rbuf, rows, acc, sem_g, sem_s):
        cid = lax.axis_index("c")
        sid = lax.axis_index("s")
        wid = cid * _NS + sid

        def zero(i, carry):
            for k in range(d // 16):
                rows[0, i, pl.ds(k * 16, 16)] = jnp.zeros((16,), jnp.float32)
            return carry

        lax.fori_loop(0, _C, zero, 0)

        def zcopy(i, carry):
            pltpu.sync_copy(rows.at[0],
                            acc.at[pl.ds(sid * spr + i * _C, _C)])
            return carry

        lax.fori_loop(0, spr // _C, zcopy, 0)
        plsc.subcore_barrier()

        mask_hi = jnp.full((16,), -65536, jnp.int32)

        def scale(b, rb):
            def body(g, carry):
                w_vec = plsc.bitcast(ibuf[b, 2, pl.ds(g * 16, 16)], jnp.float32)
                for e in range(16):
                    r = g * 16 + e
                    s = w_vec[e]
                    for k in range(dp // 16):
                        x = rbuf[rb, r, pl.ds(k * 16, 16)]
                        lo = plsc.bitcast(lax.shift_left(x, 16), jnp.float32)
                        hi = plsc.bitcast(lax.bitwise_and(x, mask_hi),
                                          jnp.float32)
                        rows[b, r, pl.ds(k * 16, 16)] = lo * s
                        rows[b, r, pl.ds(dp + k * 16, 16)] = hi * s
                return carry

            lax.fori_loop(0, _C // 16, body, 0)

        descs_s = [None] * ch
        descs_g = [None] * ch
        pltpu.sync_copy(pk_h.at[wid, 0], ibuf.at[0])
        descs_g[0] = pltpu.async_copy(tab_h.at[ibuf.at[0, 0]], rbuf.at[0],
                                      sem_g)
        for c in range(ch):
            b = c % 3
            rb = c % 2
            if c + 1 < ch:
                nb = (c + 1) % 3
                if c >= 2:
                    descs_s[c - 2].wait()
                pltpu.sync_copy(pk_h.at[wid, c + 1], ibuf.at[nb])
                descs_g[c + 1] = pltpu.async_copy(tab_h.at[ibuf.at[nb, 0]],
                                                  rbuf.at[(c + 1) % 2], sem_g)
            descs_g[c].wait()
            scale(b, rb)
            descs_s[c] = pltpu.async_copy(rows.at[b], acc.at[ibuf.at[b, 1]],
                                          sem_s, add=True)
        for t in range(min(3, ch)):
            descs_s[ch - 1 - t].wait()
        plsc.subcore_barrier()
        pltpu.sync_copy(acc.at[pl.ds(sid * spr, spr)],
                        out_h.at[cid, pl.ds(sid * spr, spr)])

    return scat(tab, pk)


_B = 2000


def _tc_pre(degt, x, w1):
    n, din = x.shape
    d = w1.shape[1]

    def body(deg_ref, x_ref, w1_ref, dis_ref, ps_ref):
        deg = deg_ref[:, 0] + deg_ref[:, 1]
        dis = jnp.where(deg > 0, lax.rsqrt(deg), 0.0)[:, None]
        dis_ref[:] = dis
        ps_ref[:] = dis * jnp.dot(x_ref[:], w1_ref[:],
                                  preferred_element_type=jnp.float32)

    return pl.pallas_call(
        body,
        grid=(n // _B,),
        in_specs=[
            pl.BlockSpec((_B, 2), lambda i: (i, 0)),
            pl.BlockSpec((_B, din), lambda i: (i, 0)),
            pl.BlockSpec((din, d), lambda i: (0, 0)),
        ],
        out_specs=[
            pl.BlockSpec((_B, 1), lambda i: (i, 0)),
            pl.BlockSpec((_B, d), lambda i: (i, 0)),
        ],
        out_shape=[
            jax.ShapeDtypeStruct((n, 1), jnp.float32),
            jax.ShapeDtypeStruct((n, d), jnp.float32),
        ],
    )(degt, x, w1)


def _tc_layer(h, a2, dis, w0, b, w1n):
    n, din = h.shape
    dout = w0.shape[1]
    ds = a2.shape[2]

    def body(h_ref, a_ref, dis_ref, w0_ref, b_ref, *rest):
        if w1n is None:
            (h1_ref, ps_ref) = rest
        else:
            (w1n_ref, h1_ref, ps_ref) = rest
        dis_b = dis_ref[:]
        m = -(dis_b * (a_ref[0, :, :] + a_ref[1, :, :]))
        z = jnp.dot(h_ref[:], w0_ref[:], preferred_element_type=jnp.float32)
        h1 = 1.0 / (1.0 + jnp.exp(-(z + m + b_ref[:])))
        h1_ref[:] = h1
        if w1n is None:
            ps_ref[:] = dis_b * h1
        else:
            ps_ref[:] = dis_b * jnp.dot(h1, w1n_ref[:],
                                        preferred_element_type=jnp.float32)

    in_specs = [
        pl.BlockSpec((_B, din), lambda i: (i, 0)),
        pl.BlockSpec((2, _B, ds), lambda i: (0, i, 0)),
        pl.BlockSpec((_B, 1), lambda i: (i, 0)),
        pl.BlockSpec((din, dout), lambda i: (0, 0)),
        pl.BlockSpec((1, dout), lambda i: (0, 0)),
    ]
    args = [h, a2, dis, w0, b.reshape(1, -1)]
    if w1n is not None:
        in_specs.append(pl.BlockSpec(w1n.shape, lambda i: (0, 0)))
        args.append(w1n)
    psd = ds if w1n is None else w1n.shape[1]
    return pl.pallas_call(
        body,
        grid=(n // _B,),
        in_specs=in_specs,
        out_specs=[
            pl.BlockSpec((_B, dout), lambda i: (i, 0)),
            pl.BlockSpec((_B, psd), lambda i: (i, 0)),
        ],
        out_shape=[
            jax.ShapeDtypeStruct((n, dout), jnp.float32),
            jax.ShapeDtypeStruct((n, psd), jnp.float32),
        ],
    )(*args)


def _tc_final(h, s2, dis, w0, b, w1):
    n, din = h.shape
    dout = w0.shape[1]
    ds = s2.shape[2]

    def body(h_ref, s_ref, dis_ref, w0_ref, b_ref, w1_ref, out_ref):
        t = -(dis_ref[:] * (s_ref[0, :, :] + s_ref[1, :, :]))
        z = (jnp.dot(h_ref[:], w0_ref[:], preferred_element_type=jnp.float32)
             + jnp.dot(t, w1_ref[:], preferred_element_type=jnp.float32)
             + b_ref[:])
        out_ref[:] = 1.0 / (1.0 + jnp.exp(-z))

    return pl.pallas_call(
        body,
        grid=(n // _B,),
        in_specs=[
            pl.BlockSpec((_B, din), lambda i: (i, 0)),
            pl.BlockSpec((2, _B, ds), lambda i: (0, i, 0)),
            pl.BlockSpec((_B, 1), lambda i: (i, 0)),
            pl.BlockSpec((din, dout), lambda i: (0, 0)),
            pl.BlockSpec((1, dout), lambda i: (0, 0)),
            pl.BlockSpec((ds, dout), lambda i: (0, 0)),
        ],
        out_specs=pl.BlockSpec((_B, dout), lambda i: (i, 0)),
        out_shape=jax.ShapeDtypeStruct((n, dout), jnp.float32),
    )(h, s2, dis, w0, b.reshape(1, -1), w1)


def kernel(x, edge_index, edge_weight,
           W0_0, W1_0, b_0,
           W0_1, W1_1, b_1,
           W0_2, W1_2, b_2,
           W0_3, W1_3, b_3):
    n = x.shape[0]
    row = edge_index[0]
    col = edge_index[1]
    row3, w3, chd, pk, ch = _pad_edges(row, col, edge_weight)
    spr = math.ceil(n / (_NS * _C)) * _C
    n_pad = spr * _NS

    def padn(t):
        bf = t.astype(jnp.bfloat16)
        d = t.shape[1]
        packed = lax.bitcast_convert_type(
            jnp.stack([bf[:, :d // 2], bf[:, d // 2:]], axis=-1), jnp.int32)
        return jnp.pad(packed, ((0, n_pad - n), (0, 0)))

    degt = _deg_kernel(row3, w3, n_pad, chd)[:, :n].T
    dis, ps = _tc_pre(degt, x, W1_0)

    a2 = _scatter_kernel(padn(ps), pk, n_pad, ch)[:, :n, :]
    h, ps = _tc_layer(x, a2, dis, W0_0, b_0, W1_1)

    a2 = _scatter_kernel(padn(ps), pk, n_pad, ch)[:, :n, :]
    h, ps = _tc_layer(h, a2, dis, W0_1, b_1, W1_2)

    a2 = _scatter_kernel(padn(ps), pk, n_pad, ch)[:, :n, :]
    h, ps = _tc_layer(h, a2, dis, W0_2, b_2, None)

    s2 = _scatter_kernel(padn(ps), pk, n_pad, ch)[:, :n, :]
    return _tc_final(h, s2, dis, W0_3, b_3, W1_3)


# speedup vs baseline: 13.0191x; 1.0867x over previous
import functools
import math

import jax
import jax.numpy as jnp
from jax import lax
from jax.experimental import pallas as pl
from jax.experimental.pallas import tpu as pltpu
from jax.experimental.pallas import tpu_sc as plsc

_NC = 2
_NS = 16
_NW = _NC * _NS
_C = 320
_CD = 640


def _sc_mesh():
    return plsc.VectorSubcoreMesh(core_axis_name="c", subcore_axis_name="s")


def _pad_edges(row, col, w):
    e = row.shape[0]
    per_w = math.ceil(e / (_NW * _CD)) * _CD
    ch = per_w // _C
    chd = per_w // _CD
    pad = per_w * _NW - e
    row = jnp.concatenate([row, jnp.zeros((pad,), jnp.int32)])
    col = jnp.concatenate([col, jnp.zeros((pad,), jnp.int32)])
    w = jnp.concatenate([w, jnp.zeros((pad,), jnp.float32)])
    pk = jnp.stack([row.reshape(_NW, ch, _C),
                    col.reshape(_NW, ch, _C),
                    lax.bitcast_convert_type(w, jnp.int32).reshape(_NW, ch, _C)],
                   axis=2)
    return (row.reshape(_NW, chd, _CD), w.reshape(_NW, chd, _CD), chd,
            pk, ch)


def _deg_kernel(row3, w3, n_pad, chd):
    spr = n_pad // _NS

    @functools.partial(
        pl.kernel,
        out_type=jax.ShapeDtypeStruct((_NC, n_pad), jnp.float32),
        mesh=_sc_mesh(),
        scratch_types=[
            pltpu.VMEM((_CD,), jnp.int32),
            pltpu.VMEM((_CD,), jnp.float32),
            pltpu.VMEM((spr,), jnp.float32),
            pltpu.VMEM_SHARED((n_pad,), jnp.float32),
        ],
        compiler_params=pltpu.CompilerParams(use_tc_tiling_on_sc=False),
    )
    def deg(row_h, w_h, out_h, rowv, wv, zv, acc):
        cid = lax.axis_index("c")
        sid = lax.axis_index("s")
        wid = cid * _NS + sid

        def zero(i, carry):
            zv[pl.ds(i * 16, 16)] = jnp.zeros((16,), jnp.float32)
            return carry

        lax.fori_loop(0, spr // 16, zero, 0)
        pltpu.sync_copy(zv, acc.at[pl.ds(sid * spr, spr)])
        plsc.subcore_barrier()

        def chunk(c, carry):
            pltpu.sync_copy(row_h.at[wid, c], rowv)
            pltpu.sync_copy(w_h.at[wid, c], wv)
            pltpu.sync_copy(wv, acc.at[rowv], add=True)
            return carry

        lax.fori_loop(0, chd, chunk, 0)
        plsc.subcore_barrier()
        pltpu.sync_copy(acc.at[pl.ds(sid * spr, spr)],
                        out_h.at[cid, pl.ds(sid * spr, spr)])

    return deg(row3, w3)


def _scatter_kernel(tab, pk, n_pad, ch):
    dp = tab.shape[1]
    d = dp * 2
    spr = n_pad // _NS

    @functools.partial(
        pl.kernel,
        out_type=jax.ShapeDtypeStruct((_NC, n_pad, d), jnp.float32),
        mesh=_sc_mesh(),
        scratch_types=[
            pltpu.VMEM((3, 3, _C), jnp.int32),
            pltpu.VMEM((2, _C, dp), jnp.int32),
            pltpu.VMEM((3, _C, d), jnp.float32),
            pltpu.VMEM_SHARED((n_pad, d), jnp.float32),
            pltpu.SemaphoreType.DMA,
            pltpu.SemaphoreType.DMA,
        ],
        compiler_params=pltpu.CompilerParams(use_tc_tiling_on_sc=False,
                                             needs_layout_passes=False),
    )
    def scat(tab_h, pk_h, out_h, ibuf, rbuf, rows, acc, sem_g, sem_s):
        cid = lax.axis_index("c")
        sid = lax.axis_index("s")
        wid = cid * _NS + sid

        def zero(i, carry):
            for k in range(d // 16):
                rows[0, i, pl.ds(k * 16, 16)] = jnp.zeros((16,), jnp.float32)
            return carry

        lax.fori_loop(0, _C, zero, 0)

        def zcopy(i, carry):
            pltpu.sync_copy(rows.at[0],
                            acc.at[pl.ds(sid * spr + i * _C, _C)])
            return carry

        lax.fori_loop(0, spr // _C, zcopy, 0)
        plsc.subcore_barrier()

        mask_hi = jnp.full((16,), -65536, jnp.int32)

        def scale(b, rb):
            def body(g, carry):
                w_vec = plsc.bitcast(ibuf[b, 2, pl.ds(g * 16, 16)], jnp.float32)
                for e in range(16):
                    r = g * 16 + e
                    s = w_vec[e]
                    for k in range(dp // 16):
                        x = rbuf[rb, r, pl.ds(k * 16, 16)]
                        lo = plsc.bitcast(lax.shift_left(x, 16), jnp.float32)
                        hi = plsc.bitcast(lax.bitwise_and(x, mask_hi),
                                          jnp.float32)
                        rows[b, r, pl.ds(k * 16, 16)] = lo * s
                        rows[b, r, pl.ds(dp + k * 16, 16)] = hi * s
                return carry

            lax.fori_loop(0, _C // 16, body, 0)

        descs_s = [None] * ch
        descs_g = [None] * ch
        pltpu.sync_copy(pk_h.at[wid, 0], ibuf.at[0])
        descs_g[0] = pltpu.async_copy(tab_h.at[ibuf.at[0, 0]], rbuf.at[0],
                                      sem_g)
        for c in range(ch):
            b = c % 3
            rb = c % 2
            if c + 1 < ch:
                nb = (c + 1) % 3
                if c >= 2:
                    descs_s[c - 2].wait()
                pltpu.sync_copy(pk_h.at[wid, c + 1], ibuf.at[nb])
                descs_g[c + 1] = pltpu.async_copy(tab_h.at[ibuf.at[nb, 0]],
                                                  rbuf.at[(c + 1) % 2], sem_g)
            descs_g[c].wait()
            scale(b, rb)
            descs_s[c] = pltpu.async_copy(rows.at[b], acc.at[ibuf.at[b, 1]],
                                          sem_s, add=True)
        for t in range(min(3, ch)):
            descs_s[ch - 1 - t].wait()
        plsc.subcore_barrier()
        pltpu.sync_copy(acc.at[pl.ds(sid * spr, spr)],
                        out_h.at[cid, pl.ds(sid * spr, spr)])

    return scat(tab, pk)


_B = 2000


def _pack_bf16(p):
    d = p.shape[1]
    u = lax.bitcast_convert_type(p.astype(jnp.bfloat16),
                                 jnp.uint16).astype(jnp.int32)
    return u[:, :d // 2] | (u[:, d // 2:] << 16)


def _tc_pre(degt, x, w1, n_pad):
    n, din = x.shape
    d = w1.shape[1]

    def body(deg_ref, x_ref, w1_ref, dis_ref, ps_ref):
        deg = deg_ref[:, 0] + deg_ref[:, 1]
        dis = jnp.where(deg > 0, lax.rsqrt(deg), 0.0)[:, None]
        dis_ref[:] = dis
        ps_ref[:] = _pack_bf16(dis * jnp.dot(x_ref[:], w1_ref[:],
                                             preferred_element_type=jnp.float32))

    return pl.pallas_call(
        body,
        grid=(n // _B,),
        in_specs=[
            pl.BlockSpec((_B, 2), lambda i: (i, 0)),
            pl.BlockSpec((_B, din), lambda i: (i, 0)),
            pl.BlockSpec((din, d), lambda i: (0, 0)),
        ],
        out_specs=[
            pl.BlockSpec((_B, 1), lambda i: (i, 0)),
            pl.BlockSpec((_B, d // 2), lambda i: (i, 0)),
        ],
        out_shape=[
            jax.ShapeDtypeStruct((n, 1), jnp.float32),
            jax.ShapeDtypeStruct((n_pad, d // 2), jnp.int32),
        ],
    )(degt, x, w1)


def _tc_layer(h, a2, dis, w0, b, w1n, n_pad):
    n, din = h.shape
    dout = w0.shape[1]
    ds = a2.shape[2]

    def body(h_ref, a_ref, dis_ref, w0_ref, b_ref, *rest):
        if w1n is None:
            (h1_ref, ps_ref) = rest
        else:
            (w1n_ref, h1_ref, ps_ref) = rest
        dis_b = dis_ref[:]
        m = -(dis_b * (a_ref[0, :, :] + a_ref[1, :, :]))
        z = jnp.dot(h_ref[:], w0_ref[:], preferred_element_type=jnp.float32)
        h1 = 1.0 / (1.0 + jnp.exp(-(z + m + b_ref[:])))
        h1_ref[:] = h1
        if w1n is None:
            ps_ref[:] = _pack_bf16(dis_b * h1)
        else:
            ps_ref[:] = _pack_bf16(
                dis_b * jnp.dot(h1, w1n_ref[:],
                                preferred_element_type=jnp.float32))

    in_specs = [
        pl.BlockSpec((_B, din), lambda i: (i, 0)),
        pl.BlockSpec((2, _B, ds), lambda i: (0, i, 0)),
        pl.BlockSpec((_B, 1), lambda i: (i, 0)),
        pl.BlockSpec((din, dout), lambda i: (0, 0)),
        pl.BlockSpec((1, dout), lambda i: (0, 0)),
    ]
    args = [h, a2, dis, w0, b.reshape(1, -1)]
    if w1n is not None:
        in_specs.append(pl.BlockSpec(w1n.shape, lambda i: (0, 0)))
        args.append(w1n)
    psd = ds if w1n is None else w1n.shape[1]
    return pl.pallas_call(
        body,
        grid=(n // _B,),
        in_specs=in_specs,
        out_specs=[
            pl.BlockSpec((_B, dout), lambda i: (i, 0)),
            pl.BlockSpec((_B, psd // 2), lambda i: (i, 0)),
        ],
        out_shape=[
            jax.ShapeDtypeStruct((n, dout), jnp.float32),
            jax.ShapeDtypeStruct((n_pad, psd // 2), jnp.int32),
        ],
    )(*args)


def _tc_final(h, s2, dis, w0, b, w1):
    n, din = h.shape
    dout = w0.shape[1]
    ds = s2.shape[2]

    def body(h_ref, s_ref, dis_ref, w0_ref, b_ref, w1_ref, out_ref):
        t = -(dis_ref[:] * (s_ref[0, :, :] + s_ref[1, :, :]))
        z = (jnp.dot(h_ref[:], w0_ref[:], preferred_element_type=jnp.float32)
             + jnp.dot(t, w1_ref[:], preferred_element_type=jnp.float32)
             + b_ref[:])
        out_ref[:] = 1.0 / (1.0 + jnp.exp(-z))

    return pl.pallas_call(
        body,
        grid=(n // _B,),
        in_specs=[
            pl.BlockSpec((_B, din), lambda i: (i, 0)),
            pl.BlockSpec((2, _B, ds), lambda i: (0, i, 0)),
            pl.BlockSpec((_B, 1), lambda i: (i, 0)),
            pl.BlockSpec((din, dout), lambda i: (0, 0)),
            pl.BlockSpec((1, dout), lambda i: (0, 0)),
            pl.BlockSpec((ds, dout), lambda i: (0, 0)),
        ],
        out_specs=pl.BlockSpec((_B, dout), lambda i: (i, 0)),
        out_shape=jax.ShapeDtypeStruct((n, dout), jnp.float32),
    )(h, s2, dis, w0, b.reshape(1, -1), w1)


def kernel(x, edge_index, edge_weight,
           W0_0, W1_0, b_0,
           W0_1, W1_1, b_1,
           W0_2, W1_2, b_2,
           W0_3, W1_3, b_3):
    n = x.shape[0]
    row = edge_index[0]
    col = edge_index[1]
    row3, w3, chd, pk, ch = _pad_edges(row, col, edge_weight)
    spr = math.ceil(n / (_NS * _C)) * _C
    n_pad = spr * _NS

    degt = _deg_kernel(row3, w3, n_pad, chd)[:, :n].T
    dis, ps = _tc_pre(degt, x, W1_0, n_pad)

    a2 = _scatter_kernel(ps, pk, n_pad, ch)
    h, ps = _tc_layer(x, a2, dis, W0_0, b_0, W1_1, n_pad)

    a2 = _scatter_kernel(ps, pk, n_pad, ch)
    h, ps = _tc_layer(h, a2, dis, W0_1, b_1, W1_2, n_pad)

    a2 = _scatter_kernel(ps, pk, n_pad, ch)
    h, ps = _tc_layer(h, a2, dis, W0_2, b_2, None, n_pad)

    s2 = _scatter_kernel(ps, pk, n_pad, ch)
    return _tc_final(h, s2, dis, W0_3, b_3, W1_3)
